# Initial kernel scaffold; baseline (speedup 1.0000x reference)
import jax
import jax.numpy as jnp
from jax.experimental import pallas as pl


def kernel(X):
    raise NotImplementedError("write your pallas kernel here")


# speedup vs baseline: 3.0497x; 3.0497x over previous
import jax
import jax.numpy as jnp
from jax import lax
from jax.experimental import pallas as pl
from jax.experimental.pallas import tpu as pltpu

_K = 512
_ITERS = 10
_CH = 1024

_DIST_PREC = lax.Precision.DEFAULT
_SEG_PREC = lax.Precision.HIGHEST


def _kmeans_kernel(x_ref, c0_ref, centers_ref, labels_ref, counts_ref,
                   stage_ref, done_ref):
    B, N, D = x_ref.shape
    K = _K
    centers_ref[...] = c0_ref[...]
    done_ref[0] = 0

    def iter_body(_, carry):
        @pl.when(done_ref[0] == 0)
        def _run():
            conv = jnp.bool_(True)
            for b in range(B):
                c = centers_ref[b]
                b2 = jnp.sum(c * c, axis=1)

                def chunk_body(nb, acc):
                    sums, counts = acc
                    xc = x_ref[b, pl.ds(nb * _CH, _CH), :]
                    ab = lax.dot_general(
                        xc, c, (((1,), (1,)), ((), ())),
                        precision=_DIST_PREC,
                        preferred_element_type=jnp.float32)
                    a2 = jnp.sum(xc * xc, axis=1)
                    d2 = jnp.maximum(a2[:, None] + b2[None, :] - 2.0 * ab, 0.0)
                    m = jnp.min(d2, axis=1)
                    kidx = lax.broadcasted_iota(jnp.int32, (_CH, K), 1)
                    lbl = jnp.min(
                        jnp.where(d2 == m[:, None], kidx, K), axis=1)
                    labels_ref[b, pl.ds(nb * _CH, _CH)] = lbl
                    onehot_t = (lax.broadcasted_iota(jnp.int32, (K, _CH), 0)
                                == lbl[None, :]).astype(jnp.float32)
                    sums = sums + lax.dot_general(
                        onehot_t, xc, (((1,), (0,)), ((), ())),
                        precision=_SEG_PREC,
                        preferred_element_type=jnp.float32)
                    counts = counts + jnp.sum(onehot_t, axis=1)
                    return sums, counts

                sums0 = jnp.zeros((K, D), jnp.float32)
                counts0 = jnp.zeros((K,), jnp.float32)
                sums, counts = lax.fori_loop(0, N // _CH, chunk_body,
                                             (sums0, counts0))
                counts_ref[b] = counts
                new_c = sums / counts[:, None]
                stage_ref[b] = new_c
                ok = jnp.abs(c - new_c) <= (1e-8 + 1e-5 * jnp.abs(new_c))
                conv = jnp.logical_and(conv, jnp.all(ok))

            @pl.when(jnp.logical_not(conv))
            def _commit():
                centers_ref[...] = stage_ref[...]

            done_ref[0] = conv.astype(jnp.int32)
        return carry

    lax.fori_loop(0, _ITERS, iter_body, 0)


def kernel(X):
    B, N, D = X.shape
    perm = jax.random.permutation(jax.random.key(42), N)[:_K]
    centers0 = X[:, perm]
    centers, labels, counts = pl.pallas_call(
        _kmeans_kernel,
        out_shape=[
            jax.ShapeDtypeStruct((B, _K, D), jnp.float32),
            jax.ShapeDtypeStruct((B, N), jnp.int32),
            jax.ShapeDtypeStruct((B, _K), jnp.float32),
        ],
        scratch_shapes=[
            pltpu.VMEM((B, _K, D), jnp.float32),
            pltpu.SMEM((1,), jnp.int32),
        ],
    )(X, centers0)
    percentages = counts / float(N)
    return centers, labels, percentages
